# baseline (speedup 1.0000x reference)
import jax
import jax.numpy as jnp
from jax import lax
from jax.experimental import pallas as pl
from jax.experimental.pallas import tpu as pltpu
from jax.experimental.pallas import tpu_sc as plsc

VOCAB = 1000000
EMB = 64
T = 200
B = 4096
NC, NS, L = 2, 16, 16
NW = NC * NS
BB = B // NW
NBUF = 4
QV = EMB // L


def _body(x_hbm, table_hbm, pos_hbm, out_hbm,
          idx_v, pos_v, oidx_v,
          buf0, buf1, buf2, buf3,
          gs0, gs1, gs2, gs3, os0, os1, os2, os3):
    bufs = (buf0, buf1, buf2, buf3)
    gsems = (gs0, gs1, gs2, gs3)
    osems = (os0, os1, os2, os3)

    wid = lax.axis_index("s") * NC + lax.axis_index("c")
    b0 = wid * BB

    pltpu.sync_copy(x_hbm.at[wid], idx_v)
    pltpu.sync_copy(pos_hbm, pos_v)

    iota = lax.iota(jnp.int32, L)
    iota_t = iota * T

    def gather_start(t, b):
        pltpu.async_copy(table_hbm.at[idx_v.at[t]], bufs[b], gsems[b])

    def gather_wait(t, b):
        pltpu.make_async_copy(table_hbm.at[idx_v.at[t]], bufs[b],
                              gsems[b]).wait()

    def scatter_start(b):
        pltpu.async_copy(bufs[b], out_hbm.at[oidx_v.at[b]], osems[b])

    def scatter_wait(b):
        pltpu.make_async_copy(bufs[b], out_hbm.at[oidx_v.at[b]],
                              osems[b]).wait()

    for b in range(NBUF - 1):
        gather_start(b, b)

    def group(g, _):
        for b in range(NBUF):
            t = g * NBUF + b
            gather_wait(t, b)

            base = b0 * T + t
            for k in range(BB // L):
                oidx_v[b, pl.ds(k * L, L)] = iota_t + (base + k * L * T)

            pvs = [pos_v[t, pl.ds(q * L, L)] for q in range(QV)]

            def row_add(i, _c, _b=b, _pvs=pvs):
                for q in range(QV):
                    sl = pl.ds(q * L, L)
                    bufs[_b][i, sl] = bufs[_b][i, sl] + _pvs[q]
                return _c

            lax.fori_loop(0, BB, row_add, 0, unroll=2)

            scatter_start(b)

            bf = (b - 1) % NBUF
            tf = t + NBUF - 1
            if b == 0:
                @pl.when(g > 0)
                def _():
                    scatter_wait(bf)
                gather_start(tf, bf)
            else:
                @pl.when(g < (T // NBUF) - 1)
                def _():
                    scatter_wait(bf)
                    gather_start(tf, bf)
        return 0

    lax.fori_loop(0, T // NBUF, group, 0)

    for b in range(NBUF):
        scatter_wait(b)


@jax.jit
def kernel(x, table, pos_emb):
    xw = x.T.reshape(T, NW, BB).transpose(1, 0, 2)
    pos = pos_emb[:T]

    kfn = pl.kernel(
        _body,
        out_type=jax.ShapeDtypeStruct((B * T, EMB), jnp.float32),
        compiler_params=pltpu.CompilerParams(use_tc_tiling_on_sc=False),
        mesh=plsc.VectorSubcoreMesh(
            core_axis_name="c", subcore_axis_name="s",
            num_cores=NC, num_subcores=NS),
        scratch_types=[
            pltpu.VMEM((T, BB), jnp.int32),
            pltpu.VMEM((T, EMB), jnp.float32),
            pltpu.VMEM((NBUF, BB), jnp.int32),
            pltpu.VMEM((BB, EMB), jnp.float32),
            pltpu.VMEM((BB, EMB), jnp.float32),
            pltpu.VMEM((BB, EMB), jnp.float32),
            pltpu.VMEM((BB, EMB), jnp.float32),
        ] + [pltpu.SemaphoreType.DMA] * (2 * NBUF),
    )
    out = kfn(xw, table, pos)
    return out.reshape(B, T, EMB)
